# Initial kernel scaffold; baseline (speedup 1.0000x reference)
import jax
import jax.numpy as jnp
from jax.experimental import pallas as pl


def kernel(inputs, edge_index, adj_values, weight, bias):
    raise NotImplementedError("write your pallas kernel here")


# speedup vs baseline: 6.1292x; 6.1292x over previous
import functools

import jax
import jax.numpy as jnp
from jax import lax
from jax.experimental import pallas as pl
from jax.experimental.pallas import tpu as pltpu
from jax.experimental.pallas import tpu_sc as plsc

EPS = 1e-5
MIN_NORM = 1e-10

CB = 80


def _pre_body(x_ref, w_ref, h_ref):
    x = x_ref[...]
    n = jnp.maximum(jnp.sqrt(jnp.sum(x * x, axis=-1, keepdims=True)), MIN_NORM)
    n_c = jnp.clip(n, MIN_NORM, 1.0 - EPS)
    at = 0.5 * jnp.log((1.0 + n_c) / (1.0 - n_c))
    t = at * x / n
    h_ref[...] = jnp.dot(t, w_ref[...], preferred_element_type=jnp.float32)


def _post_body(p_ref, b_ref, o_ref):
    agg = jnp.sum(p_ref[...], axis=0)
    n = jnp.maximum(jnp.sqrt(jnp.sum(agg * agg, -1, keepdims=True)), MIN_NORM)
    em = jnp.tanh(n) * agg / n
    n2 = jnp.maximum(jnp.sqrt(jnp.sum(em * em, -1, keepdims=True)), MIN_NORM)
    x = em * jnp.minimum(1.0, (1.0 - EPS) / n2)
    b = b_ref[...]
    nb = jnp.maximum(jnp.sqrt(jnp.sum(b * b, -1, keepdims=True)), MIN_NORM)
    bt = jnp.tanh(nb) * b / nb
    nb2 = jnp.maximum(jnp.sqrt(jnp.sum(bt * bt, -1, keepdims=True)), MIN_NORM)
    y = bt * jnp.minimum(1.0, (1.0 - EPS) / nb2)
    x2 = jnp.sum(x * x, -1, keepdims=True)
    y2 = jnp.sum(y * y, -1, keepdims=True)
    xy = jnp.sum(x * y, -1, keepdims=True)
    num = (1.0 + 2.0 * xy + y2) * x + (1.0 - x2) * y
    den = 1.0 + 2.0 * xy + x2 * y2
    den = jnp.where(jnp.abs(den) < 1e-15, 1e-15, den)
    m = num / den
    n3 = jnp.maximum(jnp.sqrt(jnp.sum(m * m, -1, keepdims=True)), MIN_NORM)
    o_ref[...] = m * jnp.minimum(1.0, (1.0 - EPS) / n3)


def _make_sc_agg(N, D, E):
    info = plsc.get_sparse_core_info()
    NC, NS = info.num_cores, info.num_subcores
    NW = NC * NS
    assert E % (NW * CB) == 0
    chunks = E // (NW * CB)
    NPAD = -(-N // (NS * 8)) * (NS * 8)
    rows_per_sub = NPAD // NS
    mesh = plsc.VectorSubcoreMesh(core_axis_name="c", subcore_axis_name="s")

    EW = chunks * CB

    @functools.partial(
        pl.kernel,
        mesh=mesh,
        out_type=jax.ShapeDtypeStruct((NC, NPAD, D), jnp.float32),
        scratch_types=[
            pltpu.VMEM((EW,), jnp.int32),
            pltpu.VMEM((EW,), jnp.int32),
            pltpu.VMEM((EW,), jnp.float32),
            pltpu.VMEM((CB, D), jnp.float32),
            pltpu.VMEM_SHARED((NPAD, D), jnp.float32),
            pltpu.SemaphoreType.DMA,
        ],
    )
    def sc_agg(h_hbm, row_hbm, col_hbm, val_hbm, zero_hbm, out_hbm,
               rowv, colv, valv, rows, acc, sem):
        c = lax.axis_index("c")
        s = lax.axis_index("s")
        wid = s * NC + c
        pltpu.sync_copy(zero_hbm.at[pl.ds(s * rows_per_sub, rows_per_sub)],
                        acc.at[pl.ds(s * rows_per_sub, rows_per_sub)])
        base = wid * EW
        pltpu.sync_copy(row_hbm.at[pl.ds(base, EW)], rowv)
        pltpu.sync_copy(col_hbm.at[pl.ds(base, EW)], colv)
        pltpu.sync_copy(val_hbm.at[pl.ds(base, EW)], valv)
        plsc.subcore_barrier()

        def chunk_body(g, carry):
            pltpu.async_copy(h_hbm.at[colv.at[pl.ds(g * CB, CB)]], rows,
                             sem).wait()

            def scale_body(q, carry2):
                vv = valv[pl.ds(g * CB + q * 16, 16)]
                for l in range(16):
                    sv = vv[l]
                    for j in range(D // 16):
                        sl = (q * 16 + l, pl.ds(16 * j, 16))
                        rows[sl] = rows[sl] * sv
                return carry2

            lax.fori_loop(0, CB // 16, scale_body, 0)
            for q in range(CB // 16):
                idx16 = rowv[pl.ds(g * CB + q * 16, 16)]
                pltpu.sync_copy(rows.at[pl.ds(q * 16, 16)], acc.at[idx16],
                                add=True)
            return carry

        lax.fori_loop(0, chunks, chunk_body, 0)
        plsc.subcore_barrier()
        pltpu.sync_copy(acc.at[pl.ds(s * rows_per_sub, rows_per_sub)],
                        out_hbm.at[c, pl.ds(s * rows_per_sub, rows_per_sub)])

    return sc_agg, NC, NPAD


def kernel(inputs, edge_index, adj_values, weight, bias):
    N, D = inputs.shape
    E = adj_values.shape[0]
    BN = 2000
    assert N % BN == 0

    h = pl.pallas_call(
        _pre_body,
        grid=(N // BN,),
        in_specs=[
            pl.BlockSpec((BN, D), lambda i: (i, 0)),
            pl.BlockSpec((D, D), lambda i: (0, 0)),
        ],
        out_specs=pl.BlockSpec((BN, D), lambda i: (i, 0)),
        out_shape=jax.ShapeDtypeStruct((N, D), jnp.float32),
    )(inputs, weight)

    sc_agg, NC, NPAD = _make_sc_agg(N, D, E)
    ei = edge_index.astype(jnp.int32)
    zeros = jnp.zeros((NPAD, D), jnp.float32)
    parts = sc_agg(h, ei[0], ei[1], adj_values, zeros)

    out = pl.pallas_call(
        _post_body,
        grid=(N // BN,),
        in_specs=[
            pl.BlockSpec((NC, BN, D), lambda i: (0, i, 0)),
            pl.BlockSpec((1, D), lambda i: (0, 0)),
        ],
        out_specs=pl.BlockSpec((BN, D), lambda i: (i, 0)),
        out_shape=jax.ShapeDtypeStruct((N, D), jnp.float32),
    )(parts, bias.reshape(1, D))
    return out
